# Initial kernel scaffold; baseline (speedup 1.0000x reference)
import jax
import jax.numpy as jnp
from jax.experimental import pallas as pl


def kernel(x, edge_index, W1, b1, W2, b2, Wd, bd):
    raise NotImplementedError("write your pallas kernel here")


# speedup vs baseline: 15.6139x; 15.6139x over previous
import functools

import jax
import jax.numpy as jnp
from jax import lax
from jax.experimental import pallas as pl
from jax.experimental.pallas import tpu as pltpu
from jax.experimental.pallas import tpu_sc as plsc

N_NODES = 10000
IN_CH = 128
HID1 = 128
HID2 = 64
OUT_CH = 128
N_EDGES = 320000
EPS = 1e-12

NC = 2
NS = 16
NW = NC * NS
NPAD = 10240
RPT = NPAD // NW
RPC = NPAD // NS
EPT = N_EDGES // NW
K = 125
NCHUNK = EPT // K
DEGW = 16
BLK = 512
GRID = NPAD // BLK


def _mesh():
    return plsc.VectorSubcoreMesh(core_axis_name="c", subcore_axis_name="s")


@functools.partial(
    pl.kernel,
    out_type=jax.ShapeDtypeStruct((NW, NPAD), jnp.float32),
    mesh=_mesh(),
    scratch_types=[
        pltpu.VMEM((EPT,), jnp.int32),
        pltpu.VMEM((NPAD,), jnp.float32),
    ],
    compiler_params=pltpu.CompilerParams(
        use_tc_tiling_on_sc=False, needs_layout_passes=False),
)
def _deg_kernel(dst_hbm, out_hbm, dst_v, cnt_v):
    cid = lax.axis_index("c")
    sid = lax.axis_index("s")
    wid = sid * NC + cid
    pltpu.sync_copy(dst_hbm.at[wid], dst_v)

    def zbody(i, carry):
        cnt_v[pl.ds(i * 16, 16)] = jnp.zeros((16,), jnp.float32)
        return carry

    lax.fori_loop(0, NPAD // 16, zbody, 0)
    ones = jnp.ones((16,), jnp.float32)

    def body(i, carry):
        idx = dst_v[pl.ds(i * 16, 16)]
        plsc.addupdate_scatter(cnt_v, [idx], ones)
        return carry

    lax.fori_loop(0, EPT // 16, body, 0)
    pltpu.sync_copy(cnt_v, out_hbm.at[wid])


def _make_mp_kernel(C):
    @functools.partial(
        pl.kernel,
        out_type=jax.ShapeDtypeStruct((NC, NPAD, C), jnp.float32),
        mesh=_mesh(),
        scratch_types=[
            pltpu.VMEM((NCHUNK, K), jnp.int32),
            pltpu.VMEM((NCHUNK, K), jnp.int32),
            pltpu.VMEM((K, C), jnp.float32),
            pltpu.VMEM((K, C), jnp.float32),
            pltpu.VMEM((RPC, C), jnp.float32),
            pltpu.VMEM_SHARED((NPAD, C), jnp.float32),
            pltpu.SemaphoreType.DMA,
            pltpu.SemaphoreType.DMA,
        ],
        compiler_params=pltpu.CompilerParams(use_tc_tiling_on_sc=False),
    )
    def mp(g_hbm, src_hbm, dst_hbm, zero_hbm, out_hbm,
           src_v, dst_v, rows0_v, rows1_v, bounce_v, acc_sh, sem0, sem1):
        cid = lax.axis_index("c")
        sid = lax.axis_index("s")
        wid = sid * NC + cid
        pltpu.sync_copy(src_hbm.at[wid], src_v)
        pltpu.sync_copy(dst_hbm.at[wid], dst_v)
        pltpu.sync_copy(zero_hbm, bounce_v)
        pltpu.sync_copy(bounce_v, acc_sh.at[pl.ds(sid * RPC, RPC)])
        plsc.subcore_barrier()

        pltpu.async_copy(g_hbm.at[src_v.at[0]], rows0_v, sem0)

        def body(jj, carry):
            j0 = 2 * jj
            j1 = j0 + 1
            pltpu.make_async_copy(g_hbm.at[src_v.at[j0]], rows0_v, sem0).wait()
            pltpu.async_copy(g_hbm.at[src_v.at[j1]], rows1_v, sem1)
            pltpu.sync_copy(rows0_v, acc_sh.at[dst_v.at[j0]], add=True)
            pltpu.make_async_copy(g_hbm.at[src_v.at[j1]], rows1_v, sem1).wait()

            @pl.when(jj < NCHUNK // 2 - 1)
            def _():
                pltpu.async_copy(g_hbm.at[src_v.at[j0 + 2]], rows0_v, sem0)

            pltpu.sync_copy(rows1_v, acc_sh.at[dst_v.at[j1]], add=True)
            return carry

        lax.fori_loop(0, NCHUNK // 2, body, 0)
        plsc.subcore_barrier()
        pltpu.sync_copy(acc_sh.at[pl.ds(sid * RPC, RPC)], bounce_v)
        pltpu.sync_copy(bounce_v, out_hbm.at[cid, pl.ds(sid * RPC, RPC)])

    return mp


_mp_kernel = _make_mp_kernel(HID2)


def _tc_a_body(x_ref, w_ref, d_ref, h_ref, ga_ref, gb_ref, dis_ref):
    h = jnp.dot(x_ref[...], w_ref[...], preferred_element_type=jnp.float32)
    deg = jnp.sum(d_ref[...], axis=0) + 1.0
    dis = lax.rsqrt(deg)
    g = h * dis
    h_ref[...] = h
    ga_ref[...] = g[:, :HID2]
    gb_ref[...] = g[:, HID2:]
    dis_ref[...] = dis


def _tc_a(xp, W1, degp):
    return pl.pallas_call(
        _tc_a_body,
        grid=(GRID,),
        in_specs=[
            pl.BlockSpec((BLK, IN_CH), lambda i: (i, 0)),
            pl.BlockSpec((IN_CH, HID1), lambda i: (0, 0)),
            pl.BlockSpec((NW, BLK, 1), lambda i: (0, i, 0)),
        ],
        out_specs=[
            pl.BlockSpec((BLK, HID1), lambda i: (i, 0)),
            pl.BlockSpec((BLK, HID2), lambda i: (i, 0)),
            pl.BlockSpec((BLK, HID2), lambda i: (i, 0)),
            pl.BlockSpec((BLK, 1), lambda i: (i, 0)),
        ],
        out_shape=[
            jax.ShapeDtypeStruct((NPAD, HID1), jnp.float32),
            jax.ShapeDtypeStruct((NPAD, HID2), jnp.float32),
            jax.ShapeDtypeStruct((NPAD, HID2), jnp.float32),
            jax.ShapeDtypeStruct((NPAD, 1), jnp.float32),
        ],
    )(xp, W1, degp)


def _tc_c_body(pa_ref, pb_ref, h1_ref, dis_ref, b1_ref, w2_ref, h2_ref, g2_ref):
    dis = dis_ref[...]
    msg = jnp.concatenate(
        [pa_ref[0] + pa_ref[1], pb_ref[0] + pb_ref[1]], axis=1)
    conv = dis * msg + (dis * dis) * h1_ref[...] + b1_ref[...]
    a = jnp.maximum(conv, 0.0)
    h2 = jnp.dot(a, w2_ref[...], preferred_element_type=jnp.float32)
    h2_ref[...] = h2
    g2_ref[...] = h2 * dis


def _tc_c(p1a, p1b, h1, dis, b1, W2):
    return pl.pallas_call(
        _tc_c_body,
        grid=(GRID,),
        in_specs=[
            pl.BlockSpec((NC, BLK, HID2), lambda i: (0, i, 0)),
            pl.BlockSpec((NC, BLK, HID2), lambda i: (0, i, 0)),
            pl.BlockSpec((BLK, HID1), lambda i: (i, 0)),
            pl.BlockSpec((BLK, 1), lambda i: (i, 0)),
            pl.BlockSpec((1, HID1), lambda i: (0, 0)),
            pl.BlockSpec((HID1, HID2), lambda i: (0, 0)),
        ],
        out_specs=[
            pl.BlockSpec((BLK, HID2), lambda i: (i, 0)),
            pl.BlockSpec((BLK, HID2), lambda i: (i, 0)),
        ],
        out_shape=[
            jax.ShapeDtypeStruct((NPAD, HID2), jnp.float32),
            jax.ShapeDtypeStruct((NPAD, HID2), jnp.float32),
        ],
    )(p1a, p1b, h1, dis, b1, W2)


def _tc_e_body(p_ref, h2_ref, dis_ref, b2_ref, wd_ref, bd_ref, lp_ref, fe_ref):
    dis = dis_ref[...]
    z = dis * (p_ref[0] + p_ref[1]) + (dis * dis) * h2_ref[...] + b2_ref[...]
    nrm = jnp.sqrt(jnp.sum(z * z, axis=1, keepdims=True))
    fe = z / (nrm + EPS)
    logits = jnp.dot(fe, wd_ref[...], preferred_element_type=jnp.float32)
    logits = logits + bd_ref[...]
    m = jnp.max(logits, axis=1, keepdims=True)
    lse = jnp.log(jnp.sum(jnp.exp(logits - m), axis=1, keepdims=True))
    lp_ref[...] = logits - m - lse
    fe_ref[...] = fe


def _tc_e(p2, h2, dis, b2, Wd, bd):
    return pl.pallas_call(
        _tc_e_body,
        grid=(GRID,),
        in_specs=[
            pl.BlockSpec((NC, BLK, HID2), lambda i: (0, i, 0)),
            pl.BlockSpec((BLK, HID2), lambda i: (i, 0)),
            pl.BlockSpec((BLK, 1), lambda i: (i, 0)),
            pl.BlockSpec((1, HID2), lambda i: (0, 0)),
            pl.BlockSpec((HID2, OUT_CH), lambda i: (0, 0)),
            pl.BlockSpec((1, OUT_CH), lambda i: (0, 0)),
        ],
        out_specs=[
            pl.BlockSpec((BLK, OUT_CH), lambda i: (i, 0)),
            pl.BlockSpec((BLK, HID2), lambda i: (i, 0)),
        ],
        out_shape=[
            jax.ShapeDtypeStruct((NPAD, OUT_CH), jnp.float32),
            jax.ShapeDtypeStruct((NPAD, HID2), jnp.float32),
        ],
    )(p2, h2, dis, b2, Wd, bd)


def kernel(x, edge_index, W1, b1, W2, b2, Wd, bd):
    ei = edge_index.astype(jnp.int32)
    src = ei[0].reshape(NW, NCHUNK, K)
    dst = ei[1].reshape(NW, NCHUNK, K)
    xp = jnp.concatenate(
        [x, jnp.zeros((NPAD - N_NODES, IN_CH), jnp.float32)], axis=0)

    zero2 = jnp.zeros((RPC, HID2), jnp.float32)

    degp = _deg_kernel(ei[1].reshape(NW, EPT))
    h1, g1a, g1b, dis = _tc_a(xp, W1, degp.reshape(NW, NPAD, 1))
    p1a = _mp_kernel(g1a, src, dst, zero2)
    p1b = _mp_kernel(g1b, src, dst, zero2)
    h2, g2 = _tc_c(p1a, p1b, h1, dis, b1.reshape(1, HID1), W2)
    p2 = _mp_kernel(g2, src, dst, zero2)
    lp, fe = _tc_e(p2, h2, dis, b2.reshape(1, HID2), Wd, bd.reshape(1, OUT_CH))
    return lp[:N_NODES], fe[:N_NODES]


# speedup vs baseline: 21.9730x; 1.4073x over previous
import functools

import jax
import jax.numpy as jnp
from jax import lax
from jax.experimental import pallas as pl
from jax.experimental.pallas import tpu as pltpu
from jax.experimental.pallas import tpu_sc as plsc

N_NODES = 10000
IN_CH = 128
HID1 = 128
HID2 = 64
OUT_CH = 128
N_EDGES = 320000
EPS = 1e-12

NC = 2
NS = 16
NW = NC * NS
NPAD = 10240
RPT = NPAD // NW
RPC = NPAD // NS
EPT = N_EDGES // NW
K = 125
NCHUNK = EPT // K
DEGW = 16
BLK = 512
GRID = NPAD // BLK


def _mesh():
    return plsc.VectorSubcoreMesh(core_axis_name="c", subcore_axis_name="s")


@functools.partial(
    pl.kernel,
    out_type=jax.ShapeDtypeStruct((NC, NPAD, DEGW), jnp.float32),
    mesh=_mesh(),
    scratch_types=[
        pltpu.VMEM((NCHUNK, K), jnp.int32),
        pltpu.VMEM((K, DEGW), jnp.float32),
        pltpu.VMEM((RPC, DEGW), jnp.float32),
        pltpu.VMEM_SHARED((NPAD, DEGW), jnp.float32),
    ],
    compiler_params=pltpu.CompilerParams(use_tc_tiling_on_sc=False),
)
def _deg_kernel(dst_hbm, ones_hbm, zero_hbm, out_hbm, dst_v, ones_v, bounce_v, acc_sh):
    cid = lax.axis_index("c")
    sid = lax.axis_index("s")
    wid = sid * NC + cid
    pltpu.sync_copy(dst_hbm.at[wid], dst_v)
    pltpu.sync_copy(ones_hbm, ones_v)
    pltpu.sync_copy(zero_hbm, bounce_v)
    pltpu.sync_copy(bounce_v, acc_sh.at[pl.ds(sid * RPC, RPC)])
    plsc.subcore_barrier()

    def body(j, carry):
        pltpu.sync_copy(ones_v, acc_sh.at[dst_v.at[j]], add=True)
        return carry

    lax.fori_loop(0, NCHUNK, body, 0)
    plsc.subcore_barrier()
    pltpu.sync_copy(acc_sh.at[pl.ds(sid * RPC, RPC)], bounce_v)
    pltpu.sync_copy(bounce_v, out_hbm.at[cid, pl.ds(sid * RPC, RPC)])


def _make_mp_kernel(C):
    @functools.partial(
        pl.kernel,
        out_type=jax.ShapeDtypeStruct((NC, NPAD, C), jnp.float32),
        mesh=_mesh(),
        scratch_types=[
            pltpu.VMEM((NCHUNK, K), jnp.int32),
            pltpu.VMEM((NCHUNK, K), jnp.int32),
            pltpu.VMEM((K, C), jnp.float32),
            pltpu.VMEM((K, C), jnp.float32),
            pltpu.VMEM((RPC, C), jnp.float32),
            pltpu.VMEM_SHARED((NPAD, C), jnp.float32),
            pltpu.SemaphoreType.DMA,
            pltpu.SemaphoreType.DMA,
        ],
        compiler_params=pltpu.CompilerParams(use_tc_tiling_on_sc=False),
    )
    def mp(g_hbm, src_hbm, dst_hbm, zero_hbm, out_hbm,
           src_v, dst_v, rows0_v, rows1_v, bounce_v, acc_sh, sem0, sem1):
        cid = lax.axis_index("c")
        sid = lax.axis_index("s")
        wid = sid * NC + cid
        pltpu.sync_copy(src_hbm.at[wid], src_v)
        pltpu.sync_copy(dst_hbm.at[wid], dst_v)
        pltpu.sync_copy(zero_hbm, bounce_v)
        pltpu.sync_copy(bounce_v, acc_sh.at[pl.ds(sid * RPC, RPC)])
        plsc.subcore_barrier()

        pltpu.async_copy(g_hbm.at[src_v.at[0]], rows0_v, sem0)

        def body(jj, carry):
            j0 = 2 * jj
            j1 = j0 + 1
            pltpu.make_async_copy(g_hbm.at[src_v.at[j0]], rows0_v, sem0).wait()
            pltpu.async_copy(g_hbm.at[src_v.at[j1]], rows1_v, sem1)
            pltpu.sync_copy(rows0_v, acc_sh.at[dst_v.at[j0]], add=True)
            pltpu.make_async_copy(g_hbm.at[src_v.at[j1]], rows1_v, sem1).wait()

            @pl.when(jj < NCHUNK // 2 - 1)
            def _():
                pltpu.async_copy(g_hbm.at[src_v.at[j0 + 2]], rows0_v, sem0)

            pltpu.sync_copy(rows1_v, acc_sh.at[dst_v.at[j1]], add=True)
            return carry

        lax.fori_loop(0, NCHUNK // 2, body, 0)
        plsc.subcore_barrier()
        pltpu.sync_copy(acc_sh.at[pl.ds(sid * RPC, RPC)], bounce_v)
        pltpu.sync_copy(bounce_v, out_hbm.at[cid, pl.ds(sid * RPC, RPC)])

    return mp


_mp_kernel = _make_mp_kernel(HID2)


def _tc_a_body(x_ref, w_ref, d_ref, h_ref, ga_ref, gb_ref, dis_ref):
    h = jnp.dot(x_ref[...], w_ref[...], preferred_element_type=jnp.float32)
    deg = d_ref[0, :, 0:1] + d_ref[1, :, 0:1] + 1.0
    dis = lax.rsqrt(deg)
    g = h * dis
    h_ref[...] = h
    ga_ref[...] = g[:, :HID2]
    gb_ref[...] = g[:, HID2:]
    dis_ref[...] = dis


def _tc_a(xp, W1, degp):
    return pl.pallas_call(
        _tc_a_body,
        grid=(GRID,),
        in_specs=[
            pl.BlockSpec((BLK, IN_CH), lambda i: (i, 0)),
            pl.BlockSpec((IN_CH, HID1), lambda i: (0, 0)),
            pl.BlockSpec((NC, BLK, DEGW), lambda i: (0, i, 0)),
        ],
        out_specs=[
            pl.BlockSpec((BLK, HID1), lambda i: (i, 0)),
            pl.BlockSpec((BLK, HID2), lambda i: (i, 0)),
            pl.BlockSpec((BLK, HID2), lambda i: (i, 0)),
            pl.BlockSpec((BLK, 1), lambda i: (i, 0)),
        ],
        out_shape=[
            jax.ShapeDtypeStruct((NPAD, HID1), jnp.float32),
            jax.ShapeDtypeStruct((NPAD, HID2), jnp.float32),
            jax.ShapeDtypeStruct((NPAD, HID2), jnp.float32),
            jax.ShapeDtypeStruct((NPAD, 1), jnp.float32),
        ],
    )(xp, W1, degp)


def _tc_c_body(pa_ref, pb_ref, h1_ref, dis_ref, b1_ref, w2_ref, h2_ref, g2_ref):
    dis = dis_ref[...]
    msg = jnp.concatenate(
        [pa_ref[0] + pa_ref[1], pb_ref[0] + pb_ref[1]], axis=1)
    conv = dis * msg + (dis * dis) * h1_ref[...] + b1_ref[...]
    a = jnp.maximum(conv, 0.0)
    h2 = jnp.dot(a, w2_ref[...], preferred_element_type=jnp.float32)
    h2_ref[...] = h2
    g2_ref[...] = h2 * dis


def _tc_c(p1a, p1b, h1, dis, b1, W2):
    return pl.pallas_call(
        _tc_c_body,
        grid=(GRID,),
        in_specs=[
            pl.BlockSpec((NC, BLK, HID2), lambda i: (0, i, 0)),
            pl.BlockSpec((NC, BLK, HID2), lambda i: (0, i, 0)),
            pl.BlockSpec((BLK, HID1), lambda i: (i, 0)),
            pl.BlockSpec((BLK, 1), lambda i: (i, 0)),
            pl.BlockSpec((1, HID1), lambda i: (0, 0)),
            pl.BlockSpec((HID1, HID2), lambda i: (0, 0)),
        ],
        out_specs=[
            pl.BlockSpec((BLK, HID2), lambda i: (i, 0)),
            pl.BlockSpec((BLK, HID2), lambda i: (i, 0)),
        ],
        out_shape=[
            jax.ShapeDtypeStruct((NPAD, HID2), jnp.float32),
            jax.ShapeDtypeStruct((NPAD, HID2), jnp.float32),
        ],
    )(p1a, p1b, h1, dis, b1, W2)


def _tc_e_body(p_ref, h2_ref, dis_ref, b2_ref, wd_ref, bd_ref, lp_ref, fe_ref):
    dis = dis_ref[...]
    z = dis * (p_ref[0] + p_ref[1]) + (dis * dis) * h2_ref[...] + b2_ref[...]
    nrm = jnp.sqrt(jnp.sum(z * z, axis=1, keepdims=True))
    fe = z / (nrm + EPS)
    logits = jnp.dot(fe, wd_ref[...], preferred_element_type=jnp.float32)
    logits = logits + bd_ref[...]
    m = jnp.max(logits, axis=1, keepdims=True)
    lse = jnp.log(jnp.sum(jnp.exp(logits - m), axis=1, keepdims=True))
    lp_ref[...] = logits - m - lse
    fe_ref[...] = fe


def _tc_e(p2, h2, dis, b2, Wd, bd):
    return pl.pallas_call(
        _tc_e_body,
        grid=(GRID,),
        in_specs=[
            pl.BlockSpec((NC, BLK, HID2), lambda i: (0, i, 0)),
            pl.BlockSpec((BLK, HID2), lambda i: (i, 0)),
            pl.BlockSpec((BLK, 1), lambda i: (i, 0)),
            pl.BlockSpec((1, HID2), lambda i: (0, 0)),
            pl.BlockSpec((HID2, OUT_CH), lambda i: (0, 0)),
            pl.BlockSpec((1, OUT_CH), lambda i: (0, 0)),
        ],
        out_specs=[
            pl.BlockSpec((BLK, OUT_CH), lambda i: (i, 0)),
            pl.BlockSpec((BLK, HID2), lambda i: (i, 0)),
        ],
        out_shape=[
            jax.ShapeDtypeStruct((NPAD, OUT_CH), jnp.float32),
            jax.ShapeDtypeStruct((NPAD, HID2), jnp.float32),
        ],
    )(p2, h2, dis, b2, Wd, bd)


def kernel(x, edge_index, W1, b1, W2, b2, Wd, bd):
    ei = edge_index.astype(jnp.int32)
    src = ei[0].reshape(NW, NCHUNK, K)
    dst = ei[1].reshape(NW, NCHUNK, K)
    xp = jnp.concatenate(
        [x, jnp.zeros((NPAD - N_NODES, IN_CH), jnp.float32)], axis=0)

    zero2 = jnp.zeros((RPC, HID2), jnp.float32)

    ones_deg = jnp.ones((K, DEGW), jnp.float32)
    zero_deg = jnp.zeros((RPC, DEGW), jnp.float32)
    degp = _deg_kernel(dst, ones_deg, zero_deg)
    h1, g1a, g1b, dis = _tc_a(xp, W1, degp)
    p1a = _mp_kernel(g1a, src, dst, zero2)
    p1b = _mp_kernel(g1b, src, dst, zero2)
    h2, g2 = _tc_c(p1a, p1b, h1, dis, b1.reshape(1, HID1), W2)
    p2 = _mp_kernel(g2, src, dst, zero2)
    lp, fe = _tc_e(p2, h2, dis, b2.reshape(1, HID2), Wd, bd.reshape(1, OUT_CH))
    return lp[:N_NODES], fe[:N_NODES]


# speedup vs baseline: 25.8738x; 1.1775x over previous
import functools

import jax
import jax.numpy as jnp
from jax import lax
from jax.experimental import pallas as pl
from jax.experimental.pallas import tpu as pltpu
from jax.experimental.pallas import tpu_sc as plsc

N_NODES = 10000
IN_CH = 128
HID1 = 128
HID2 = 64
OUT_CH = 128
N_EDGES = 320000
EPS = 1e-12

NC = 2
NS = 16
NW = NC * NS
NPAD = 10240
RPT = NPAD // NW
RPC = NPAD // NS
EPT = N_EDGES // NW
K = 125
NCHUNK = EPT // K
DEGW = 16
BLK = 512
GRID = NPAD // BLK


def _mesh():
    return plsc.VectorSubcoreMesh(core_axis_name="c", subcore_axis_name="s")


@functools.partial(
    pl.kernel,
    out_type=jax.ShapeDtypeStruct((NC, NPAD, DEGW), jnp.float32),
    mesh=_mesh(),
    scratch_types=[
        pltpu.VMEM((NCHUNK, K), jnp.int32),
        pltpu.VMEM((K, DEGW), jnp.float32),
        pltpu.VMEM((RPC, DEGW), jnp.float32),
        pltpu.VMEM_SHARED((NPAD, DEGW), jnp.float32),
    ],
    compiler_params=pltpu.CompilerParams(use_tc_tiling_on_sc=False),
)
def _deg_kernel(dst_hbm, ones_hbm, zero_hbm, out_hbm, dst_v, ones_v, bounce_v, acc_sh):
    cid = lax.axis_index("c")
    sid = lax.axis_index("s")
    wid = sid * NC + cid
    pltpu.sync_copy(dst_hbm.at[wid], dst_v)
    pltpu.sync_copy(ones_hbm, ones_v)
    pltpu.sync_copy(zero_hbm, bounce_v)
    pltpu.sync_copy(bounce_v, acc_sh.at[pl.ds(sid * RPC, RPC)])
    plsc.subcore_barrier()

    def body(j, carry):
        pltpu.sync_copy(ones_v, acc_sh.at[dst_v.at[j]], add=True)
        return carry

    lax.fori_loop(0, NCHUNK, body, 0)
    plsc.subcore_barrier()
    pltpu.sync_copy(acc_sh.at[pl.ds(sid * RPC, RPC)], bounce_v)
    pltpu.sync_copy(bounce_v, out_hbm.at[cid, pl.ds(sid * RPC, RPC)])


NB = 3


def _make_mp_kernel(C):
    @functools.partial(
        pl.kernel,
        out_type=jax.ShapeDtypeStruct((NC, NPAD, C), jnp.float32),
        mesh=_mesh(),
        scratch_types=[
            pltpu.VMEM((NCHUNK, K), jnp.int32),
            pltpu.VMEM((NCHUNK, K), jnp.int32),
            [pltpu.VMEM((K, C), jnp.float32)] * NB,
            pltpu.VMEM((RPC, C), jnp.float32),
            pltpu.VMEM_SHARED((NPAD, C), jnp.float32),
            [pltpu.SemaphoreType.DMA] * NB,
            [pltpu.SemaphoreType.DMA] * NB,
        ],
        compiler_params=pltpu.CompilerParams(use_tc_tiling_on_sc=False),
    )
    def mp(g_hbm, src_hbm, dst_hbm, zero_hbm, out_hbm,
           src_v, dst_v, rows_v, bounce_v, acc_sh, sem_g, sem_s):
        cid = lax.axis_index("c")
        sid = lax.axis_index("s")
        wid = sid * NC + cid
        pltpu.sync_copy(src_hbm.at[wid], src_v)
        pltpu.sync_copy(dst_hbm.at[wid], dst_v)
        pltpu.sync_copy(zero_hbm, bounce_v)
        pltpu.sync_copy(bounce_v, acc_sh.at[pl.ds(sid * RPC, RPC)])
        plsc.subcore_barrier()

        for b in range(NB):
            pltpu.async_copy(g_hbm.at[src_v.at[b]], rows_v[b], sem_g[b])

        def body(jj, carry):
            base = NB * jj
            for b in range(NB):
                j = base + b

                @pl.when(j < NCHUNK)
                def _(b=b, j=j):
                    pltpu.make_async_copy(
                        g_hbm.at[src_v.at[j]], rows_v[b], sem_g[b]).wait()
                    pltpu.async_copy(
                        rows_v[b], acc_sh.at[dst_v.at[j]], sem_s[b], add=True)

            for b in range(NB):
                j = base + b

                @pl.when(j < NCHUNK)
                def _(b=b, j=j):
                    pltpu.make_async_copy(
                        rows_v[b], acc_sh.at[dst_v.at[j]], sem_s[b]).wait()

                @pl.when(j + NB < NCHUNK)
                def _(b=b, j=j):
                    pltpu.async_copy(
                        g_hbm.at[src_v.at[j + NB]], rows_v[b], sem_g[b])

            return carry

        lax.fori_loop(0, -(-NCHUNK // NB), body, 0)
        plsc.subcore_barrier()
        pltpu.sync_copy(acc_sh.at[pl.ds(sid * RPC, RPC)], bounce_v)
        pltpu.sync_copy(bounce_v, out_hbm.at[cid, pl.ds(sid * RPC, RPC)])

    return mp


_mp_kernel = _make_mp_kernel(HID2)


def _tc_a_body(x_ref, w_ref, d_ref, h_ref, ga_ref, gb_ref, dis_ref):
    h = jnp.dot(x_ref[...], w_ref[...], preferred_element_type=jnp.float32)
    deg = d_ref[0, :, 0:1] + d_ref[1, :, 0:1] + 1.0
    dis = lax.rsqrt(deg)
    g = h * dis
    h_ref[...] = h
    ga_ref[...] = g[:, :HID2]
    gb_ref[...] = g[:, HID2:]
    dis_ref[...] = dis


def _tc_a(xp, W1, degp):
    return pl.pallas_call(
        _tc_a_body,
        grid=(GRID,),
        in_specs=[
            pl.BlockSpec((BLK, IN_CH), lambda i: (i, 0)),
            pl.BlockSpec((IN_CH, HID1), lambda i: (0, 0)),
            pl.BlockSpec((NC, BLK, DEGW), lambda i: (0, i, 0)),
        ],
        out_specs=[
            pl.BlockSpec((BLK, HID1), lambda i: (i, 0)),
            pl.BlockSpec((BLK, HID2), lambda i: (i, 0)),
            pl.BlockSpec((BLK, HID2), lambda i: (i, 0)),
            pl.BlockSpec((BLK, 1), lambda i: (i, 0)),
        ],
        out_shape=[
            jax.ShapeDtypeStruct((NPAD, HID1), jnp.float32),
            jax.ShapeDtypeStruct((NPAD, HID2), jnp.float32),
            jax.ShapeDtypeStruct((NPAD, HID2), jnp.float32),
            jax.ShapeDtypeStruct((NPAD, 1), jnp.float32),
        ],
    )(xp, W1, degp)


def _tc_c_body(pa_ref, pb_ref, h1_ref, dis_ref, b1_ref, w2_ref, h2_ref, g2_ref):
    dis = dis_ref[...]
    msg = jnp.concatenate(
        [pa_ref[0] + pa_ref[1], pb_ref[0] + pb_ref[1]], axis=1)
    conv = dis * msg + (dis * dis) * h1_ref[...] + b1_ref[...]
    a = jnp.maximum(conv, 0.0)
    h2 = jnp.dot(a, w2_ref[...], preferred_element_type=jnp.float32)
    h2_ref[...] = h2
    g2_ref[...] = h2 * dis


def _tc_c(p1a, p1b, h1, dis, b1, W2):
    return pl.pallas_call(
        _tc_c_body,
        grid=(GRID,),
        in_specs=[
            pl.BlockSpec((NC, BLK, HID2), lambda i: (0, i, 0)),
            pl.BlockSpec((NC, BLK, HID2), lambda i: (0, i, 0)),
            pl.BlockSpec((BLK, HID1), lambda i: (i, 0)),
            pl.BlockSpec((BLK, 1), lambda i: (i, 0)),
            pl.BlockSpec((1, HID1), lambda i: (0, 0)),
            pl.BlockSpec((HID1, HID2), lambda i: (0, 0)),
        ],
        out_specs=[
            pl.BlockSpec((BLK, HID2), lambda i: (i, 0)),
            pl.BlockSpec((BLK, HID2), lambda i: (i, 0)),
        ],
        out_shape=[
            jax.ShapeDtypeStruct((NPAD, HID2), jnp.float32),
            jax.ShapeDtypeStruct((NPAD, HID2), jnp.float32),
        ],
    )(p1a, p1b, h1, dis, b1, W2)


def _tc_e_body(p_ref, h2_ref, dis_ref, b2_ref, wd_ref, bd_ref, lp_ref, fe_ref):
    dis = dis_ref[...]
    z = dis * (p_ref[0] + p_ref[1]) + (dis * dis) * h2_ref[...] + b2_ref[...]
    nrm = jnp.sqrt(jnp.sum(z * z, axis=1, keepdims=True))
    fe = z / (nrm + EPS)
    logits = jnp.dot(fe, wd_ref[...], preferred_element_type=jnp.float32)
    logits = logits + bd_ref[...]
    m = jnp.max(logits, axis=1, keepdims=True)
    lse = jnp.log(jnp.sum(jnp.exp(logits - m), axis=1, keepdims=True))
    lp_ref[...] = logits - m - lse
    fe_ref[...] = fe


def _tc_e(p2, h2, dis, b2, Wd, bd):
    return pl.pallas_call(
        _tc_e_body,
        grid=(GRID,),
        in_specs=[
            pl.BlockSpec((NC, BLK, HID2), lambda i: (0, i, 0)),
            pl.BlockSpec((BLK, HID2), lambda i: (i, 0)),
            pl.BlockSpec((BLK, 1), lambda i: (i, 0)),
            pl.BlockSpec((1, HID2), lambda i: (0, 0)),
            pl.BlockSpec((HID2, OUT_CH), lambda i: (0, 0)),
            pl.BlockSpec((1, OUT_CH), lambda i: (0, 0)),
        ],
        out_specs=[
            pl.BlockSpec((BLK, OUT_CH), lambda i: (i, 0)),
            pl.BlockSpec((BLK, HID2), lambda i: (i, 0)),
        ],
        out_shape=[
            jax.ShapeDtypeStruct((NPAD, OUT_CH), jnp.float32),
            jax.ShapeDtypeStruct((NPAD, HID2), jnp.float32),
        ],
    )(p2, h2, dis, b2, Wd, bd)


def kernel(x, edge_index, W1, b1, W2, b2, Wd, bd):
    ei = edge_index.astype(jnp.int32)
    src = ei[0].reshape(NW, NCHUNK, K)
    dst = ei[1].reshape(NW, NCHUNK, K)
    xp = jnp.concatenate(
        [x, jnp.zeros((NPAD - N_NODES, IN_CH), jnp.float32)], axis=0)

    zero2 = jnp.zeros((RPC, HID2), jnp.float32)

    ones_deg = jnp.ones((K, DEGW), jnp.float32)
    zero_deg = jnp.zeros((RPC, DEGW), jnp.float32)
    degp = _deg_kernel(dst, ones_deg, zero_deg)
    h1, g1a, g1b, dis = _tc_a(xp, W1, degp)
    p1a = _mp_kernel(g1a, src, dst, zero2)
    p1b = _mp_kernel(g1b, src, dst, zero2)
    h2, g2 = _tc_c(p1a, p1b, h1, dis, b1.reshape(1, HID1), W2)
    p2 = _mp_kernel(g2, src, dst, zero2)
    lp, fe = _tc_e(p2, h2, dis, b2.reshape(1, HID2), Wd, bd.reshape(1, OUT_CH))
    return lp[:N_NODES], fe[:N_NODES]
